# baseline (speedup 1.0000x reference)
import jax
import jax.numpy as jnp
from jax import lax
from jax.experimental import pallas as pl
from jax.experimental.pallas import tpu as pltpu
from jax.experimental.pallas import tpu_sc as plsc

B = 16384
NUM_CAT = 26
NUM_NUM = 13
N = NUM_CAT + NUM_NUM
DIM = 128

NC = 2
NS = 16
NW = NC * NS
ROWS_W = B // NW
CHUNK = 8
NCHUNK = ROWS_W // CHUNK


def _body(idx_hbm, num_hbm, table_hbm, w_hbm, b_hbm, out_hbm,
          num_v, w_v, b_v, slab0, slab1, ib0, ib1, gsem, osem, isem):
    wid = lax.axis_index("s") * NC + lax.axis_index("c")

    pltpu.sync_copy(num_hbm.at[pl.ds(wid * ROWS_W * NUM_NUM, ROWS_W * NUM_NUM)],
                    num_v.at[pl.ds(0, ROWS_W * NUM_NUM)])
    pltpu.sync_copy(w_hbm, w_v)
    pltpu.sync_copy(b_hbm, b_v)

    wk = [w_v[pl.ds(16 * k, 16)] for k in range(DIM // 16)]
    bk = [b_v[pl.ds(16 * k, 16)] for k in range(DIM // 16)]

    def idx_copy(c, ib):
        return pltpu.make_async_copy(
            idx_hbm.at[pl.ds(wid * ROWS_W + c * CHUNK, CHUNK)], ib, isem)

    def gather_copies(ib, sl):
        return [
            pltpu.make_async_copy(table_hbm.at[ib.at[r]], sl.at[r], gsem)
            for r in range(CHUNK)
        ]

    def out_copy(c, sl):
        return pltpu.make_async_copy(
            sl, out_hbm.at[pl.ds(wid * ROWS_W + c * CHUNK, CHUNK)], osem)

    def compute_num(c, sl):
        for r in range(CHUNK):
            def num_body(j, carry2, r=r):
                v = num_v[pl.ds((c * CHUNK + r) * NUM_NUM + j, 16)]
                splat = jnp.full((16,), v[0], jnp.float32)
                for k in range(DIM // 16):
                    sl[r, NUM_CAT + j, pl.ds(16 * k, 16)] = jnp.maximum(
                        splat * wk[k] + bk[k], 0.0)
                return carry2

            lax.fori_loop(0, NUM_NUM, num_body, 0)

    def when(cond, fn):
        if cond is None:
            fn()
        else:
            pl.when(cond)(fn)

    def half(c, sl, sl_other, ib, ib_other, drain_out_c, has_drain, has_next):
        when(has_next, lambda: idx_copy(c + 1, ib_other).start())
        for g in gather_copies(ib, sl):
            g.wait()
        when(has_drain, lambda: out_copy(drain_out_c, sl_other).wait())

        def _issue():
            idx_copy(c + 1, ib_other).wait()
            for g in gather_copies(ib_other, sl_other):
                g.start()

        when(has_next, _issue)
        compute_num(c, sl)
        out_copy(c, sl).start()

    idx_copy(0, ib0).start()
    idx_copy(0, ib0).wait()
    for g in gather_copies(ib0, slab0):
        g.start()

    def super_body(t, carry):
        a = 2 * t
        half(a, slab0, slab1, ib0, ib1, a - 1, t > 0, None)
        half(a + 1, slab1, slab0, ib1, ib0, a, None, t < NCHUNK // 2 - 1)
        return carry

    lax.fori_loop(0, NCHUNK // 2, super_body, 0)
    out_copy(NCHUNK - 1, slab1).wait()


def kernel(x, emb_table, W, b):
    idx39 = jnp.concatenate(
        [x[:, :NUM_CAT].astype(jnp.int32),
         jnp.zeros((B, NUM_NUM), jnp.int32)], axis=1)
    num = x[:, NUM_CAT:].reshape(B * NUM_NUM)
    w_flat = W.reshape(DIM)

    f = pl.kernel(
        _body,
        out_type=jax.ShapeDtypeStruct((B, N, DIM), jnp.float32),
        mesh=plsc.VectorSubcoreMesh(core_axis_name="c", subcore_axis_name="s"),
        scratch_types=[
            pltpu.VMEM((ROWS_W * NUM_NUM + 16,), jnp.float32),
            pltpu.VMEM((DIM,), jnp.float32),
            pltpu.VMEM((DIM,), jnp.float32),
            pltpu.VMEM((CHUNK, N, DIM), jnp.float32),
            pltpu.VMEM((CHUNK, N, DIM), jnp.float32),
            pltpu.VMEM((CHUNK, N), jnp.int32),
            pltpu.VMEM((CHUNK, N), jnp.int32),
            pltpu.SemaphoreType.DMA,
            pltpu.SemaphoreType.DMA,
            pltpu.SemaphoreType.DMA,
        ],
    )
    return f(idx39, num, emb_table, w_flat, b)


# speedup vs baseline: 1.0022x; 1.0022x over previous
import jax
import jax.numpy as jnp
from jax import lax
from jax.experimental import pallas as pl
from jax.experimental.pallas import tpu as pltpu
from jax.experimental.pallas import tpu_sc as plsc

B = 16384
NUM_CAT = 26
NUM_NUM = 13
N = NUM_CAT + NUM_NUM
DIM = 128

NC = 2
NS = 16
NW = NC * NS
ROWS_W = B // NW
CHUNK = 8
NCHUNK = ROWS_W // CHUNK


def _body(idx_hbm, num_hbm, table_hbm, w_hbm, b_hbm, out_hbm,
          num_v, w_v, b_v, slab0, slab1, ib0, ib1, gsem, osem, isem):
    wid = lax.axis_index("s") * NC + lax.axis_index("c")

    pltpu.sync_copy(num_hbm.at[pl.ds(wid * ROWS_W * NUM_NUM, ROWS_W * NUM_NUM)],
                    num_v.at[pl.ds(0, ROWS_W * NUM_NUM)])
    pltpu.sync_copy(w_hbm, w_v)
    pltpu.sync_copy(b_hbm, b_v)

    wk = [w_v[pl.ds(16 * k, 16)] for k in range(DIM // 16)]
    bk = [b_v[pl.ds(16 * k, 16)] for k in range(DIM // 16)]

    def idx_copy(c, ib):
        return pltpu.make_async_copy(
            idx_hbm.at[pl.ds(wid * ROWS_W + c * CHUNK, CHUNK)], ib, isem)

    def gather_copies(ib, sl):
        return [
            pltpu.make_async_copy(table_hbm.at[ib.at[r]], sl.at[r], gsem)
            for r in range(CHUNK)
        ]

    def out_copies(c, sl):
        return [
            pltpu.make_async_copy(
                sl.at[r], out_hbm.at[wid * ROWS_W + c * CHUNK + r], osem)
            for r in range(CHUNK)
        ]

    def compute_num(c, sl):
        for r in range(CHUNK):
            def num_body(j, carry2, r=r):
                v = num_v[pl.ds((c * CHUNK + r) * NUM_NUM + j, 16)]
                splat = jnp.full((16,), v[0], jnp.float32)
                for k in range(DIM // 16):
                    sl[r, NUM_CAT + j, pl.ds(16 * k, 16)] = jnp.maximum(
                        splat * wk[k] + bk[k], 0.0)
                return carry2

            lax.fori_loop(0, NUM_NUM, num_body, 0)

    def when(cond, fn):
        if cond is None:
            fn()
        else:
            pl.when(cond)(fn)

    def half(c, sl, sl_other, ib, ib_other, drain_out_c, has_drain, has_next):
        when(has_next, lambda: idx_copy(c + 1, ib_other).start())
        for g in gather_copies(ib, sl):
            g.wait()
        def _drain():
            for oc in out_copies(drain_out_c, sl_other):
                oc.wait()

        when(has_drain, _drain)

        def _issue():
            idx_copy(c + 1, ib_other).wait()
            for g in gather_copies(ib_other, sl_other):
                g.start()

        when(has_next, _issue)
        compute_num(c, sl)
        for oc in out_copies(c, sl):
            oc.start()

    idx_copy(0, ib0).start()
    idx_copy(0, ib0).wait()
    for g in gather_copies(ib0, slab0):
        g.start()

    def super_body(t, carry):
        a = 2 * t
        half(a, slab0, slab1, ib0, ib1, a - 1, t > 0, None)
        half(a + 1, slab1, slab0, ib1, ib0, a, None, t < NCHUNK // 2 - 1)
        return carry

    lax.fori_loop(0, NCHUNK // 2, super_body, 0)
    for oc in out_copies(NCHUNK - 1, slab1):
        oc.wait()


def kernel(x, emb_table, W, b):
    idx39 = jnp.concatenate(
        [x[:, :NUM_CAT].astype(jnp.int32),
         jnp.zeros((B, NUM_NUM), jnp.int32)], axis=1)
    num = x[:, NUM_CAT:].reshape(B * NUM_NUM)
    w_flat = W.reshape(DIM)

    f = pl.kernel(
        _body,
        out_type=jax.ShapeDtypeStruct((B, N, DIM), jnp.float32),
        mesh=plsc.VectorSubcoreMesh(core_axis_name="c", subcore_axis_name="s"),
        scratch_types=[
            pltpu.VMEM((ROWS_W * NUM_NUM + 16,), jnp.float32),
            pltpu.VMEM((DIM,), jnp.float32),
            pltpu.VMEM((DIM,), jnp.float32),
            pltpu.VMEM((CHUNK, N, DIM), jnp.float32),
            pltpu.VMEM((CHUNK, N, DIM), jnp.float32),
            pltpu.VMEM((CHUNK, N), jnp.int32),
            pltpu.VMEM((CHUNK, N), jnp.int32),
            pltpu.SemaphoreType.DMA,
            pltpu.SemaphoreType.DMA,
            pltpu.SemaphoreType.DMA,
        ],
    )
    return f(idx39, num, emb_table, w_flat, b)


# speedup vs baseline: 11.7160x; 11.6906x over previous
import jax
import jax.numpy as jnp
from jax import lax
from jax.experimental import pallas as pl
from jax.experimental.pallas import tpu as pltpu
from jax.experimental.pallas import tpu_sc as plsc

B = 16384
NUM_CAT = 26
NUM_NUM = 13
N = NUM_CAT + NUM_NUM
DIM = 128

NC = 2
NS = 16
NW = NC * NS
ROWS_W = B // NW
CHUNK = 8
NCHUNK = ROWS_W // CHUNK
SLAB = CHUNK * N


def _body(cat_hbm, num_hbm, table_hbm, w_hbm, b_hbm, out_hbm,
          num_v, w_v, b_v, slab0, slab1, ib0, ib1, gsem, osem, isem):
    wid = lax.axis_index("s") * NC + lax.axis_index("c")

    pltpu.sync_copy(num_hbm.at[pl.ds(wid * ROWS_W * NUM_NUM, ROWS_W * NUM_NUM)],
                    num_v.at[pl.ds(0, ROWS_W * NUM_NUM)])
    pltpu.sync_copy(w_hbm, w_v)
    pltpu.sync_copy(b_hbm, b_v)

    wk = [w_v[pl.ds(16 * k, 16)] for k in range(DIM // 16)]
    bk = [b_v[pl.ds(16 * k, 16)] for k in range(DIM // 16)]

    def idx_copy(c, ib):
        return pltpu.make_async_copy(
            cat_hbm.at[pl.ds(wid * ROWS_W + c * CHUNK, CHUNK)], ib, isem)

    def gather_copies(ib, sl):
        return [
            pltpu.make_async_copy(table_hbm.at[ib.at[r]],
                                  sl.at[pl.ds(r * N, NUM_CAT)], gsem)
            for r in range(CHUNK)
        ]

    def out_copies(c, sl):
        return [
            pltpu.make_async_copy(
                sl.at[pl.ds(r * N, N)],
                out_hbm.at[wid * ROWS_W + c * CHUNK + r], osem)
            for r in range(CHUNK)
        ]

    def compute_num(c, sl):
        for r in range(CHUNK):
            def num_body(j, carry2, r=r):
                v = num_v[pl.ds((c * CHUNK + r) * NUM_NUM + j, 16)]
                splat = jnp.full((16,), v[0], jnp.float32)
                for k in range(DIM // 16):
                    sl[r * N + NUM_CAT + j, pl.ds(16 * k, 16)] = jnp.maximum(
                        splat * wk[k] + bk[k], 0.0)
                return carry2

            lax.fori_loop(0, NUM_NUM, num_body, 0)

    def when(cond, fn):
        if cond is None:
            fn()
        else:
            pl.when(cond)(fn)

    def half(c, sl, sl_other, ib, ib_other, drain_out_c, has_drain, has_next):
        when(has_next, lambda: idx_copy(c + 1, ib_other).start())
        compute_num(c, sl)
        for g in gather_copies(ib, sl):
            g.wait()

        def _drain():
            for oc in out_copies(drain_out_c, sl_other):
                oc.wait()

        when(has_drain, _drain)

        def _issue():
            idx_copy(c + 1, ib_other).wait()
            for g in gather_copies(ib_other, sl_other):
                g.start()

        when(has_next, _issue)
        for oc in out_copies(c, sl):
            oc.start()

    idx_copy(0, ib0).start()
    idx_copy(0, ib0).wait()
    for g in gather_copies(ib0, slab0):
        g.start()

    def super_body(t, carry):
        a = 2 * t
        half(a, slab0, slab1, ib0, ib1, a - 1, t > 0, None)
        half(a + 1, slab1, slab0, ib1, ib0, a, None, t < NCHUNK // 2 - 1)
        return carry

    lax.fori_loop(0, NCHUNK // 2, super_body, 0)
    for oc in out_copies(NCHUNK - 1, slab1):
        oc.wait()


def kernel(x, emb_table, W, b):
    cat_idx = x[:, :NUM_CAT].astype(jnp.int32)
    num = x[:, NUM_CAT:].reshape(B * NUM_NUM)
    w_flat = W.reshape(DIM)

    f = pl.kernel(
        _body,
        out_type=jax.ShapeDtypeStruct((B, N, DIM), jnp.float32),
        mesh=plsc.VectorSubcoreMesh(core_axis_name="c", subcore_axis_name="s"),
        scratch_types=[
            pltpu.VMEM((ROWS_W * NUM_NUM + 16,), jnp.float32),
            pltpu.VMEM((DIM,), jnp.float32),
            pltpu.VMEM((DIM,), jnp.float32),
            pltpu.VMEM((SLAB, DIM), jnp.float32),
            pltpu.VMEM((SLAB, DIM), jnp.float32),
            pltpu.VMEM((CHUNK, NUM_CAT), jnp.int32),
            pltpu.VMEM((CHUNK, NUM_CAT), jnp.int32),
            pltpu.SemaphoreType.DMA,
            pltpu.SemaphoreType.DMA,
            pltpu.SemaphoreType.DMA,
        ],
    )
    return f(cat_idx, num, emb_table, w_flat, b)


# speedup vs baseline: 25.3270x; 2.1617x over previous
import jax
import jax.numpy as jnp
from jax import lax
from jax.experimental import pallas as pl
from jax.experimental.pallas import tpu as pltpu
from jax.experimental.pallas import tpu_sc as plsc

B = 16384
NUM_CAT = 26
NUM_NUM = 13
N = NUM_CAT + NUM_NUM
DIM = 128

NC = 2
NS = 16
NW = NC * NS
ROWS_W = B // NW
CHUNK = 8
NCHUNK = ROWS_W // CHUNK
SLAB = CHUNK * N


def _body(cat_hbm, num_hbm, table_hbm, w_hbm, b_hbm, out_hbm,
          num_v, w_v, b_v, table_v, slab0, slab1, ib0, ib1, gsem, osem, isem):
    wid = lax.axis_index("s") * NC + lax.axis_index("c")

    pltpu.sync_copy(num_hbm.at[pl.ds(wid * ROWS_W * NUM_NUM, ROWS_W * NUM_NUM)],
                    num_v.at[pl.ds(0, ROWS_W * NUM_NUM)])
    pltpu.sync_copy(w_hbm, w_v)
    pltpu.sync_copy(b_hbm, b_v)

    @pl.when(lax.axis_index("s") == 0)
    def _stage_table():
        pltpu.sync_copy(table_hbm, table_v)

    plsc.subcore_barrier()

    wk = [w_v[pl.ds(16 * k, 16)] for k in range(DIM // 16)]
    bk = [b_v[pl.ds(16 * k, 16)] for k in range(DIM // 16)]

    def idx_copy(c, ib):
        return pltpu.make_async_copy(
            cat_hbm.at[pl.ds(wid * ROWS_W + c * CHUNK, CHUNK)], ib, isem)

    def gather_copies(ib, sl):
        return [
            pltpu.make_async_copy(table_v.at[ib.at[r]],
                                  sl.at[pl.ds(r * N, NUM_CAT)], gsem)
            for r in range(CHUNK)
        ]

    def out_copies(c, sl):
        return [
            pltpu.make_async_copy(
                sl.at[pl.ds(r * N, N)],
                out_hbm.at[wid * ROWS_W + c * CHUNK + r], osem)
            for r in range(CHUNK)
        ]

    def compute_num(c, sl):
        def row_body(r, carry2):
            v = num_v[pl.ds((c * CHUNK + r) * NUM_NUM, 16)]
            for j in range(NUM_NUM):
                splat = jnp.full((16,), v[j], jnp.float32)
                base = r * N + NUM_CAT + j
                for k in range(DIM // 16):
                    sl[base, pl.ds(16 * k, 16)] = jnp.maximum(
                        splat * wk[k] + bk[k], 0.0)
            return carry2

        lax.fori_loop(0, CHUNK, row_body, 0)

    def when(cond, fn):
        if cond is None:
            fn()
        else:
            pl.when(cond)(fn)

    def half(c, sl, sl_other, ib, ib_other, drain_out_c, has_drain, has_next):
        when(has_next, lambda: idx_copy(c + 1, ib_other).start())
        compute_num(c, sl)
        for g in gather_copies(ib, sl):
            g.wait()

        def _drain():
            for oc in out_copies(drain_out_c, sl_other):
                oc.wait()

        when(has_drain, _drain)

        def _issue():
            idx_copy(c + 1, ib_other).wait()
            for g in gather_copies(ib_other, sl_other):
                g.start()

        when(has_next, _issue)
        for oc in out_copies(c, sl):
            oc.start()

    idx_copy(0, ib0).start()
    idx_copy(0, ib0).wait()
    for g in gather_copies(ib0, slab0):
        g.start()

    def super_body(t, carry):
        a = 2 * t
        half(a, slab0, slab1, ib0, ib1, a - 1, t > 0, None)
        half(a + 1, slab1, slab0, ib1, ib0, a, None, t < NCHUNK // 2 - 1)
        return carry

    lax.fori_loop(0, NCHUNK // 2, super_body, 0)
    for oc in out_copies(NCHUNK - 1, slab1):
        oc.wait()


def kernel(x, emb_table, W, b):
    cat_idx = x[:, :NUM_CAT].astype(jnp.int32)
    num = x[:, NUM_CAT:].reshape(B * NUM_NUM)
    w_flat = W.reshape(DIM)

    f = pl.kernel(
        _body,
        out_type=jax.ShapeDtypeStruct((B, N, DIM), jnp.float32),
        mesh=plsc.VectorSubcoreMesh(core_axis_name="c", subcore_axis_name="s"),
        scratch_types=[
            pltpu.VMEM((ROWS_W * NUM_NUM + 16,), jnp.float32),
            pltpu.VMEM((DIM,), jnp.float32),
            pltpu.VMEM((DIM,), jnp.float32),
            pltpu.VMEM_SHARED((2 * DIM, DIM), jnp.float32),
            pltpu.VMEM((SLAB, DIM), jnp.float32),
            pltpu.VMEM((SLAB, DIM), jnp.float32),
            pltpu.VMEM((CHUNK, NUM_CAT), jnp.int32),
            pltpu.VMEM((CHUNK, NUM_CAT), jnp.int32),
            pltpu.SemaphoreType.DMA,
            pltpu.SemaphoreType.DMA,
            pltpu.SemaphoreType.DMA,
        ],
    )
    return f(cat_idx, num, emb_table, w_flat, b)
